# Initial kernel scaffold; baseline (speedup 1.0000x reference)
import jax
import jax.numpy as jnp
from jax.experimental import pallas as pl


def kernel(group, period, ls, Wg, Wp, Wl):
    raise NotImplementedError("write your pallas kernel here")


# speedup vs baseline: 1.3239x; 1.3239x over previous
import functools

import jax
import jax.numpy as jnp
from jax import lax
from jax.experimental import pallas as pl
from jax.experimental.pallas import tpu as pltpu
from jax.experimental.pallas import tpu_sc as plsc

N = 100000
DIM = 256
DG = 128
R = 128
NB = (N + R - 1) // R
NB_FULL = N // R
TAIL = N - NB_FULL * R
NW = 32
MAX_ITERS = (NB_FULL + NW - 1) // NW
W_TAIL = NB_FULL % NW
L = 16


def _body(g_idx_h, p_idx_h, l_idx_h, wg_h, wpl_h, out_h,
          idx_g, idx_p, idx_l, idx_pl, rows, sem):
    c = lax.axis_index("c")
    s = lax.axis_index("s")
    w = s * 2 + c

    def do_block(b, nrows):
        pltpu.sync_copy(g_idx_h.at[b], idx_g)
        pltpu.sync_copy(p_idx_h.at[b], idx_p)
        pltpu.sync_copy(l_idx_h.at[b], idx_l)
        for k in range(R // L):
            sl = pl.ds(k * L, L)
            idx_pl[sl] = idx_p[sl] * 3 + idx_l[sl]
        h1 = pltpu.async_copy(wg_h.at[idx_g], rows.at[:, pl.ds(0, DG)], sem)
        h2 = pltpu.async_copy(wpl_h.at[idx_pl], rows.at[:, pl.ds(DG, DG)],
                              sem)
        h1.wait()
        h2.wait()
        base = b * R
        pltpu.sync_copy(rows.at[pl.ds(0, nrows)],
                        out_h.at[pl.ds(base, nrows), :])

    def loop_body(i, carry):
        b = w + i * NW

        @pl.when(b < NB_FULL)
        def _():
            do_block(b, R)

        return carry

    lax.fori_loop(0, MAX_ITERS, loop_body, 0)

    @pl.when(w == W_TAIL)
    def _():
        do_block(NB_FULL, TAIL)


@jax.jit
def kernel(group, period, ls, Wg, Wp, Wl):
    pad = NB * R - N
    g2 = jnp.pad(group.astype(jnp.int32), (0, pad)).reshape(NB, R)
    p2 = jnp.pad(period.astype(jnp.int32), (0, pad)).reshape(NB, R)
    l2 = jnp.pad(ls.astype(jnp.int32), (0, pad)).reshape(NB, R)
    wpl = jnp.concatenate(
        [jnp.repeat(Wp, 3, axis=0), jnp.tile(Wl, (Wp.shape[0], 1))], axis=1)

    mesh = plsc.VectorSubcoreMesh(core_axis_name="c", subcore_axis_name="s")
    run = functools.partial(
        pl.kernel,
        mesh=mesh,
        out_type=jax.ShapeDtypeStruct((N, DIM), jnp.float32),
        scratch_types=[
            pltpu.VMEM((R,), jnp.int32),
            pltpu.VMEM((R,), jnp.int32),
            pltpu.VMEM((R,), jnp.int32),
            pltpu.VMEM((R,), jnp.int32),
            pltpu.VMEM((R, DIM), jnp.float32),
            pltpu.SemaphoreType.DMA,
        ],
    )(_body)
    return run(g2, p2, l2, Wg, wpl)


# speedup vs baseline: 1.4599x; 1.1027x over previous
import functools

import jax
import jax.numpy as jnp
from jax import lax
from jax.experimental import pallas as pl
from jax.experimental.pallas import tpu as pltpu
from jax.experimental.pallas import tpu_sc as plsc

N = 100000
DIM = 256
DG = 128
R = 128
NB = (N + R - 1) // R
NW = 32
BPW = 25
SLAB = BPW * R
L = 16


def _body(g_h, p_h, l_h, wg_h, wpl_h, out_h,
          idx_g, idx_p, idx_l, idx_pl, rows0, rows1,
          sem_i, sg0, sg1, sw0, sw1):
    c = lax.axis_index("c")
    s = lax.axis_index("s")
    w = s * 2 + c
    start = (w * (NB - BPW)) // (NW - 1)
    e0 = start * R

    h1 = pltpu.async_copy(g_h.at[pl.ds(e0, SLAB)], idx_g, sem_i)
    h2 = pltpu.async_copy(p_h.at[pl.ds(e0, SLAB)], idx_p, sem_i)
    h3 = pltpu.async_copy(l_h.at[pl.ds(e0, SLAB)], idx_l, sem_i)
    h1.wait()
    h2.wait()
    h3.wait()

    for k in range(SLAB // L):
        sl = pl.ds(k * L, L)
        idx_pl[sl] = idx_p[sl] * 3 + idx_l[sl]

    bufs = (rows0, rows1)
    sgs = (sg0, sg1)
    sws = (sw0, sw1)

    def fire_gathers(j):
        slot = j % 2
        buf = bufs[slot]
        isl = pl.ds(j * R, R)
        a = pltpu.async_copy(wg_h.at[idx_g.at[isl]],
                             buf.at[:, pl.ds(0, DG)], sgs[slot])
        b = pltpu.async_copy(wpl_h.at[idx_pl.at[isl]],
                             buf.at[:, pl.ds(DG, DG)], sgs[slot])
        return a, b

    def fire_write(j):
        slot = j % 2
        base = jnp.minimum((start + j) * R, N - R)
        return pltpu.async_copy(bufs[slot], out_h.at[pl.ds(base, R), :],
                                sws[slot])

    gh = [None] * BPW
    wh = [None] * BPW
    for j in range(BPW):
        if j >= 2:
            wh[j - 2].wait()
        gh[j] = fire_gathers(j)
        if j >= 1:
            gh[j - 1][0].wait()
            gh[j - 1][1].wait()
            wh[j - 1] = fire_write(j - 1)
    gh[BPW - 1][0].wait()
    gh[BPW - 1][1].wait()
    wh[BPW - 1] = fire_write(BPW - 1)
    wh[BPW - 2].wait()
    wh[BPW - 1].wait()


@jax.jit
def kernel(group, period, ls, Wg, Wp, Wl):
    def layout(x):
        x = x.astype(jnp.int32)
        return jnp.concatenate([x[:(NB - 1) * R], x[N - R:]])

    g1 = layout(group)
    p1 = layout(period)
    l1 = layout(ls)
    wpl = jnp.concatenate(
        [jnp.repeat(Wp, 3, axis=0), jnp.tile(Wl, (Wp.shape[0], 1))], axis=1)

    mesh = plsc.VectorSubcoreMesh(core_axis_name="c", subcore_axis_name="s")
    run = functools.partial(
        pl.kernel,
        mesh=mesh,
        out_type=jax.ShapeDtypeStruct((N, DIM), jnp.float32),
        scratch_types=[
            pltpu.VMEM((SLAB,), jnp.int32),
            pltpu.VMEM((SLAB,), jnp.int32),
            pltpu.VMEM((SLAB,), jnp.int32),
            pltpu.VMEM((SLAB,), jnp.int32),
            pltpu.VMEM((R, DIM), jnp.float32),
            pltpu.VMEM((R, DIM), jnp.float32),
            pltpu.SemaphoreType.DMA,
            pltpu.SemaphoreType.DMA,
            pltpu.SemaphoreType.DMA,
            pltpu.SemaphoreType.DMA,
            pltpu.SemaphoreType.DMA,
        ],
    )(_body)
    return run(g1, p1, l1, Wg, wpl)


# speedup vs baseline: 4.5317x; 3.1041x over previous
import functools

import jax
import jax.numpy as jnp
from jax import lax
from jax.experimental import pallas as pl
from jax.experimental.pallas import tpu as pltpu
from jax.experimental.pallas import tpu_sc as plsc

N = 100000
DIM = 256
DG = 128
R = 128
NB = (N + R - 1) // R
NW = 32
BPW = 25
SLAB = BPW * R
L = 16


def _body(g_h, p_h, l_h, wf_h, out_h,
          idx_g, idx_p, idx_l, idx_pl, rows0, rows1,
          sem_i, sg0, sg1, sw0, sw1):
    c = lax.axis_index("c")
    s = lax.axis_index("s")
    w = s * 2 + c
    start = (w * (NB - BPW)) // (NW - 1)
    e0 = start * R

    h1 = pltpu.async_copy(g_h.at[pl.ds(e0, SLAB)], idx_g, sem_i)
    h2 = pltpu.async_copy(p_h.at[pl.ds(e0, SLAB)], idx_p, sem_i)
    h3 = pltpu.async_copy(l_h.at[pl.ds(e0, SLAB)], idx_l, sem_i)
    h1.wait()
    h2.wait()
    h3.wait()

    for k in range(SLAB // L):
        sl = pl.ds(k * L, L)
        idx_pl[sl] = idx_g[sl] * 21 + idx_p[sl] * 3 + idx_l[sl]

    bufs = (rows0, rows1)
    sgs = (sg0, sg1)
    sws = (sw0, sw1)

    def fire_gathers(j):
        slot = j % 2
        isl = pl.ds(j * R, R)
        a = pltpu.async_copy(wf_h.at[idx_pl.at[isl]], bufs[slot], sgs[slot])
        return (a,)

    def fire_write(j):
        slot = j % 2
        base = jnp.minimum((start + j) * R, N - R)
        return pltpu.async_copy(bufs[slot], out_h.at[pl.ds(base, R), :],
                                sws[slot])

    gh = [None] * BPW
    wh = [None] * BPW
    for j in range(BPW):
        if j >= 2:
            wh[j - 2].wait()
        gh[j] = fire_gathers(j)
        if j >= 1:
            gh[j - 1][0].wait()
            wh[j - 1] = fire_write(j - 1)
    gh[BPW - 1][0].wait()
    wh[BPW - 1] = fire_write(BPW - 1)
    wh[BPW - 2].wait()
    wh[BPW - 1].wait()


@jax.jit
def kernel(group, period, ls, Wg, Wp, Wl):
    def layout(x):
        x = x.astype(jnp.int32)
        return jnp.concatenate([x[:(NB - 1) * R], x[N - R:]])

    g1 = layout(group)
    p1 = layout(period)
    l1 = layout(ls)
    wf = jnp.concatenate([
        jnp.repeat(Wg, 21, axis=0),
        jnp.tile(jnp.repeat(Wp, 3, axis=0), (Wg.shape[0], 1)),
        jnp.tile(Wl, (Wg.shape[0] * Wp.shape[0], 1)),
    ], axis=1)

    mesh = plsc.VectorSubcoreMesh(core_axis_name="c", subcore_axis_name="s")
    run = functools.partial(
        pl.kernel,
        mesh=mesh,
        out_type=jax.ShapeDtypeStruct((N, DIM), jnp.float32),
        scratch_types=[
            pltpu.VMEM((SLAB,), jnp.int32),
            pltpu.VMEM((SLAB,), jnp.int32),
            pltpu.VMEM((SLAB,), jnp.int32),
            pltpu.VMEM((SLAB,), jnp.int32),
            pltpu.VMEM((R, DIM), jnp.float32),
            pltpu.VMEM((R, DIM), jnp.float32),
            pltpu.SemaphoreType.DMA,
            pltpu.SemaphoreType.DMA,
            pltpu.SemaphoreType.DMA,
            pltpu.SemaphoreType.DMA,
            pltpu.SemaphoreType.DMA,
        ],
    )(_body)
    return run(g1, p1, l1, wf)
